# baseline (speedup 1.0000x reference)
import functools

import jax
import jax.numpy as jnp
from jax import lax
from jax.experimental import pallas as pl
from jax.experimental.pallas import tpu as pltpu
from jax.experimental.pallas import tpu_sc as plsc

_VOCAB = 100000
_DIM = 64
_B = 4096
_L = 200
_TAGS = 59
_TDIM = 4
_EDIM = _TDIM + 2
_EPAD = 8
_ODIM = _DIM + _EDIM
_EXT = _TAGS * 4

_N = _B * _L
_LANES = 128
_ROWS = _N // _LANES
_NC = 2
_NS = 16
_NW = _NC * _NS
_ROWS_PER_W = _ROWS // _NW
_CHUNK_ROWS = 4
_CHUNK = _CHUNK_ROWS * _LANES
_STEPS = _ROWS_PER_W // _CHUNK_ROWS


def _sc_gather(word_table, ext_table, idx_all):
    mesh = plsc.VectorSubcoreMesh(core_axis_name="c", subcore_axis_name="s")

    @functools.partial(
        pl.kernel,
        mesh=mesh,
        compiler_params=pltpu.CompilerParams(use_tc_tiling_on_sc=False),
        out_type=jax.ShapeDtypeStruct((_N, _ODIM), jnp.float32),
        scratch_types=[
            pltpu.VMEM((_CHUNK_ROWS, 2, _LANES), jnp.int32),
            pltpu.VMEM((_CHUNK_ROWS, 2, _LANES), jnp.int32),
            pltpu.VMEM((_CHUNK, _DIM), jnp.float32),
            pltpu.VMEM((_CHUNK, _DIM), jnp.float32),
            pltpu.VMEM((_CHUNK, _EPAD), jnp.float32),
            pltpu.VMEM((_CHUNK, _EPAD), jnp.float32),
            pltpu.SemaphoreType.DMA,
            pltpu.SemaphoreType.DMA,
            pltpu.SemaphoreType.DMA,
            pltpu.SemaphoreType.DMA,
        ],
    )
    def k(word_hbm, ext_hbm, idx_hbm, out_hbm,
          ibuf0, ibuf1, wbuf0, wbuf1, ebuf0, ebuf1,
          gsem0, gsem1, ssem0, ssem1):
        wid = lax.axis_index("s") * _NC + lax.axis_index("c")
        row0 = wid * _ROWS_PER_W
        ibuf = (ibuf0, ibuf1)
        wbuf = (wbuf0, wbuf1)
        ebuf = (ebuf0, ebuf1)
        gsem = (gsem0, gsem1)
        ssem = (ssem0, ssem1)

        def gather_copies(p):
            for b in range(_CHUNK_ROWS):
                yield pltpu.make_async_copy(
                    word_hbm.at[ibuf[p].at[b, 0]],
                    wbuf[p].at[pl.ds(b * _LANES, _LANES)],
                    gsem[p])
                yield pltpu.make_async_copy(
                    ext_hbm.at[ibuf[p].at[b, 1]],
                    ebuf[p].at[pl.ds(b * _LANES, _LANES)],
                    gsem[p])

        def store_copies(p, r):
            base = r * _LANES
            yield pltpu.make_async_copy(
                wbuf[p],
                out_hbm.at[pl.ds(base, _CHUNK), pl.ds(0, _DIM)],
                ssem[p])
            yield pltpu.make_async_copy(
                ebuf[p].at[:, pl.ds(0, _EDIM)],
                out_hbm.at[pl.ds(base, _CHUNK), pl.ds(_DIM, _EDIM)],
                ssem[p])

        pltpu.sync_copy(idx_hbm.at[pl.ds(row0, _CHUNK_ROWS)], ibuf[0])
        for c in gather_copies(0):
            c.start()
        pltpu.sync_copy(
            idx_hbm.at[pl.ds(row0 + _CHUNK_ROWS, _CHUNK_ROWS)], ibuf[1])

        def step(j, p):
            r = row0 + j * _CHUNK_ROWS

            @pl.when(j >= 1)
            def _():
                for c in store_copies(1 - p, r):
                    c.wait()

            for c in gather_copies(p):
                c.wait()

            @pl.when(j + 1 < _STEPS)
            def _():
                for c in gather_copies(1 - p):
                    c.start()

            for c in store_copies(p, r):
                c.start()

            @pl.when(j + 2 < _STEPS)
            def _():
                pltpu.sync_copy(
                    idx_hbm.at[pl.ds(r + 2 * _CHUNK_ROWS, _CHUNK_ROWS)],
                    ibuf[p])

        def body(i, carry):
            step(2 * i, 0)
            step(2 * i + 1, 1)
            return carry

        lax.fori_loop(0, _STEPS // 2, body, 0)

        for c in store_copies(1, row0 + (_STEPS - 1) * _CHUNK_ROWS):
            c.wait()

    return k(word_table, ext_table, idx_all)


def _tc_prep(data_0, data_1, data_2, data_3):
    bb = 128

    def body(d0, d1, d2, d3, o):
        e = d1[...] * 4 + d2[...] * 2 + d3[...]
        o[:, 0, 0, :] = jnp.transpose(d0[...], (1, 0))
        o[:, 0, 1, :] = jnp.transpose(e, (1, 0))

    out = pl.pallas_call(
        body,
        grid=(_B // bb,),
        in_specs=[pl.BlockSpec((bb, _L), lambda j: (j, 0))] * 4,
        out_specs=pl.BlockSpec((_L, 1, 2, _LANES), lambda j: (0, j, 0, 0)),
        out_shape=jax.ShapeDtypeStruct((_L, _B // bb, 2, _LANES), jnp.int32),
    )(data_0, data_1, data_2, data_3)
    return out.reshape(_ROWS, 2, _LANES)


def kernel(data_0, data_1, data_2, data_3, word_table, tag_table, is_content):
    s = jnp.asarray(is_content, jnp.float32)
    idx_all = _tc_prep(data_0, data_1, data_2, data_3)
    e = jnp.arange(_EXT, dtype=jnp.int32)
    ext = jnp.concatenate([
        jnp.repeat(tag_table, 4, axis=0),
        (((e >> 1) & 1).astype(jnp.float32) * s)[:, None],
        ((e & 1).astype(jnp.float32) * s)[:, None],
        jnp.zeros((_EXT, _EPAD - _EDIM), jnp.float32),
    ], axis=1)
    out = _sc_gather(word_table, ext, idx_all)
    return out.reshape(_L, _B, _ODIM)


# speedup vs baseline: 1.0231x; 1.0231x over previous
import functools

import jax
import jax.numpy as jnp
from jax import lax
from jax.experimental import pallas as pl
from jax.experimental.pallas import tpu as pltpu
from jax.experimental.pallas import tpu_sc as plsc

_VOCAB = 100000
_DIM = 64
_B = 4096
_L = 200
_TAGS = 59
_TDIM = 4
_EDIM = _TDIM + 2
_EPAD = 8
_ODIM = _DIM + _EDIM
_EXT = _TAGS * 4

_N = _B * _L
_LANES = 256
_ROWS = _N // _LANES
_NC = 2
_NS = 16
_NW = _NC * _NS
_ROWS_PER_W = _ROWS // _NW
_CHUNK_ROWS = 2
_CHUNK = _CHUNK_ROWS * _LANES
_STEPS = _ROWS_PER_W // _CHUNK_ROWS


def _sc_gather(word_table, ext_table, idx_all):
    mesh = plsc.VectorSubcoreMesh(core_axis_name="c", subcore_axis_name="s")

    @functools.partial(
        pl.kernel,
        mesh=mesh,
        compiler_params=pltpu.CompilerParams(use_tc_tiling_on_sc=False),
        out_type=jax.ShapeDtypeStruct((_N, _ODIM), jnp.float32),
        scratch_types=[
            pltpu.VMEM((_CHUNK_ROWS, 2, _LANES), jnp.int32),
            pltpu.VMEM((_CHUNK_ROWS, 2, _LANES), jnp.int32),
            pltpu.VMEM((_CHUNK, _DIM), jnp.float32),
            pltpu.VMEM((_CHUNK, _DIM), jnp.float32),
            pltpu.VMEM((_CHUNK, _EPAD), jnp.float32),
            pltpu.VMEM((_CHUNK, _EPAD), jnp.float32),
            pltpu.SemaphoreType.DMA,
            pltpu.SemaphoreType.DMA,
            pltpu.SemaphoreType.DMA,
            pltpu.SemaphoreType.DMA,
        ],
    )
    def k(word_hbm, ext_hbm, idx_hbm, out_hbm,
          ibuf0, ibuf1, wbuf0, wbuf1, ebuf0, ebuf1,
          gsem0, gsem1, ssem0, ssem1):
        wid = lax.axis_index("s") * _NC + lax.axis_index("c")
        row0 = wid * _ROWS_PER_W
        ibuf = (ibuf0, ibuf1)
        wbuf = (wbuf0, wbuf1)
        ebuf = (ebuf0, ebuf1)
        gsem = (gsem0, gsem1)
        ssem = (ssem0, ssem1)

        def gather_copies(p):
            for b in range(_CHUNK_ROWS):
                yield pltpu.make_async_copy(
                    word_hbm.at[ibuf[p].at[b, 0]],
                    wbuf[p].at[pl.ds(b * _LANES, _LANES)],
                    gsem[p])
                yield pltpu.make_async_copy(
                    ext_hbm.at[ibuf[p].at[b, 1]],
                    ebuf[p].at[pl.ds(b * _LANES, _LANES)],
                    gsem[p])

        def store_copies(p, r):
            base = r * _LANES
            yield pltpu.make_async_copy(
                wbuf[p],
                out_hbm.at[pl.ds(base, _CHUNK), pl.ds(0, _DIM)],
                ssem[p])
            yield pltpu.make_async_copy(
                ebuf[p].at[:, pl.ds(0, _EDIM)],
                out_hbm.at[pl.ds(base, _CHUNK), pl.ds(_DIM, _EDIM)],
                ssem[p])

        pltpu.sync_copy(idx_hbm.at[pl.ds(row0, _CHUNK_ROWS)], ibuf[0])
        for c in gather_copies(0):
            c.start()
        pltpu.sync_copy(
            idx_hbm.at[pl.ds(row0 + _CHUNK_ROWS, _CHUNK_ROWS)], ibuf[1])

        def step(j, p):
            r = row0 + j * _CHUNK_ROWS

            @pl.when(j >= 1)
            def _():
                for c in store_copies(1 - p, r):
                    c.wait()

            for c in gather_copies(p):
                c.wait()

            @pl.when(j + 1 < _STEPS)
            def _():
                for c in gather_copies(1 - p):
                    c.start()

            for c in store_copies(p, r):
                c.start()

            @pl.when(j + 2 < _STEPS)
            def _():
                pltpu.sync_copy(
                    idx_hbm.at[pl.ds(r + 2 * _CHUNK_ROWS, _CHUNK_ROWS)],
                    ibuf[p])

        def body(i, carry):
            step(2 * i, 0)
            step(2 * i + 1, 1)
            return carry

        lax.fori_loop(0, _STEPS // 2, body, 0)

        for c in store_copies(1, row0 + (_STEPS - 1) * _CHUNK_ROWS):
            c.wait()

    return k(word_table, ext_table, idx_all)


def _tc_prep(data_0, data_1, data_2, data_3):
    bb = _LANES

    def body(d0, d1, d2, d3, o):
        e = d1[...] * 4 + d2[...] * 2 + d3[...]
        o[:, 0, 0, :] = jnp.transpose(d0[...], (1, 0))
        o[:, 0, 1, :] = jnp.transpose(e, (1, 0))

    out = pl.pallas_call(
        body,
        grid=(_B // bb,),
        in_specs=[pl.BlockSpec((bb, _L), lambda j: (j, 0))] * 4,
        out_specs=pl.BlockSpec((_L, 1, 2, _LANES), lambda j: (0, j, 0, 0)),
        out_shape=jax.ShapeDtypeStruct((_L, _B // bb, 2, _LANES), jnp.int32),
    )(data_0, data_1, data_2, data_3)
    return out.reshape(_ROWS, 2, _LANES)


def kernel(data_0, data_1, data_2, data_3, word_table, tag_table, is_content):
    s = jnp.asarray(is_content, jnp.float32)
    idx_all = _tc_prep(data_0, data_1, data_2, data_3)
    e = jnp.arange(_EXT, dtype=jnp.int32)
    ext = jnp.concatenate([
        jnp.repeat(tag_table, 4, axis=0),
        (((e >> 1) & 1).astype(jnp.float32) * s)[:, None],
        ((e & 1).astype(jnp.float32) * s)[:, None],
        jnp.zeros((_EXT, _EPAD - _EDIM), jnp.float32),
    ], axis=1)
    out = _sc_gather(word_table, ext, idx_all)
    return out.reshape(_L, _B, _ODIM)


# speedup vs baseline: 1.2906x; 1.2614x over previous
import functools

import jax
import jax.numpy as jnp
from jax import lax
from jax.experimental import pallas as pl
from jax.experimental.pallas import tpu as pltpu
from jax.experimental.pallas import tpu_sc as plsc

_VOCAB = 100000
_DIM = 64
_B = 4096
_L = 200
_TAGS = 59
_TDIM = 4
_EDIM = _TDIM + 2
_EPAD = 8
_ODIM = _DIM + _EDIM
_EXT = _TAGS * 4

_N = _B * _L
_LANES = 256
_ROWS = _N // _LANES
_NC = 2
_NS = 16
_NW = _NC * _NS
_ROWS_PER_W = _ROWS // _NW
_CHUNK_ROWS = 2
_CHUNK = _CHUNK_ROWS * _LANES
_STEPS = _ROWS_PER_W // _CHUNK_ROWS


def _sc_gather(word_table, ext_table, idx_all):
    mesh = plsc.VectorSubcoreMesh(core_axis_name="c", subcore_axis_name="s")

    @functools.partial(
        pl.kernel,
        mesh=mesh,
        compiler_params=pltpu.CompilerParams(use_tc_tiling_on_sc=False),
        out_type=[jax.ShapeDtypeStruct((_N, _DIM), jnp.float32),
                  jax.ShapeDtypeStruct((_N, _EPAD), jnp.float32)],
        scratch_types=[
            pltpu.VMEM((_CHUNK_ROWS, 2, _LANES), jnp.int32),
            pltpu.VMEM((_CHUNK_ROWS, 2, _LANES), jnp.int32),
            pltpu.VMEM((_CHUNK, _DIM), jnp.float32),
            pltpu.VMEM((_CHUNK, _DIM), jnp.float32),
            pltpu.VMEM((_CHUNK, _EPAD), jnp.float32),
            pltpu.VMEM((_CHUNK, _EPAD), jnp.float32),
            pltpu.SemaphoreType.DMA,
            pltpu.SemaphoreType.DMA,
            pltpu.SemaphoreType.DMA,
            pltpu.SemaphoreType.DMA,
        ],
    )
    def k(word_hbm, ext_hbm, idx_hbm, outw_hbm, oute_hbm,
          ibuf0, ibuf1, wbuf0, wbuf1, ebuf0, ebuf1,
          gsem0, gsem1, ssem0, ssem1):
        wid = lax.axis_index("s") * _NC + lax.axis_index("c")
        row0 = wid * _ROWS_PER_W
        ibuf = (ibuf0, ibuf1)
        wbuf = (wbuf0, wbuf1)
        ebuf = (ebuf0, ebuf1)
        gsem = (gsem0, gsem1)
        ssem = (ssem0, ssem1)

        def gather_copies(p):
            for b in range(_CHUNK_ROWS):
                yield pltpu.make_async_copy(
                    word_hbm.at[ibuf[p].at[b, 0]],
                    wbuf[p].at[pl.ds(b * _LANES, _LANES)],
                    gsem[p])
                yield pltpu.make_async_copy(
                    ext_hbm.at[ibuf[p].at[b, 1]],
                    ebuf[p].at[pl.ds(b * _LANES, _LANES)],
                    gsem[p])

        def store_copies(p, r):
            base = r * _LANES
            yield pltpu.make_async_copy(
                wbuf[p], outw_hbm.at[pl.ds(base, _CHUNK)], ssem[p])
            yield pltpu.make_async_copy(
                ebuf[p], oute_hbm.at[pl.ds(base, _CHUNK)], ssem[p])

        pltpu.sync_copy(idx_hbm.at[pl.ds(row0, _CHUNK_ROWS)], ibuf[0])
        for c in gather_copies(0):
            c.start()
        pltpu.sync_copy(
            idx_hbm.at[pl.ds(row0 + _CHUNK_ROWS, _CHUNK_ROWS)], ibuf[1])

        def step(j, p):
            r = row0 + j * _CHUNK_ROWS

            @pl.when(j >= 1)
            def _():
                for c in store_copies(1 - p, r):
                    c.wait()

            for c in gather_copies(p):
                c.wait()

            @pl.when(j + 1 < _STEPS)
            def _():
                for c in gather_copies(1 - p):
                    c.start()

            for c in store_copies(p, r):
                c.start()

            @pl.when(j + 2 < _STEPS)
            def _():
                pltpu.sync_copy(
                    idx_hbm.at[pl.ds(r + 2 * _CHUNK_ROWS, _CHUNK_ROWS)],
                    ibuf[p])

        def body(i, carry):
            step(2 * i, 0)
            step(2 * i + 1, 1)
            return carry

        lax.fori_loop(0, _STEPS // 2, body, 0)

        for c in store_copies(1, row0 + (_STEPS - 1) * _CHUNK_ROWS):
            c.wait()

    return k(word_table, ext_table, idx_all)


def _tc_prep(data_0, data_1, data_2, data_3):
    bb = _LANES

    def body(d0, d1, d2, d3, o):
        e = d1[...] * 4 + d2[...] * 2 + d3[...]
        o[:, 0, 0, :] = jnp.transpose(d0[...], (1, 0))
        o[:, 0, 1, :] = jnp.transpose(e, (1, 0))

    out = pl.pallas_call(
        body,
        grid=(_B // bb,),
        in_specs=[pl.BlockSpec((bb, _L), lambda j: (j, 0))] * 4,
        out_specs=pl.BlockSpec((_L, 1, 2, _LANES), lambda j: (0, j, 0, 0)),
        out_shape=jax.ShapeDtypeStruct((_L, _B // bb, 2, _LANES), jnp.int32),
    )(data_0, data_1, data_2, data_3)
    return out.reshape(_ROWS, 2, _LANES)


def kernel(data_0, data_1, data_2, data_3, word_table, tag_table, is_content):
    s = jnp.asarray(is_content, jnp.float32)
    idx_all = _tc_prep(data_0, data_1, data_2, data_3)
    e = jnp.arange(_EXT, dtype=jnp.int32)
    ext = jnp.concatenate([
        jnp.repeat(tag_table, 4, axis=0),
        (((e >> 1) & 1).astype(jnp.float32) * s)[:, None],
        ((e & 1).astype(jnp.float32) * s)[:, None],
        jnp.zeros((_EXT, _EPAD - _EDIM), jnp.float32),
    ], axis=1)
    out_w, out_e = _sc_gather(word_table, ext, idx_all)
    return jnp.concatenate([
        out_w.reshape(_L, _B, _DIM),
        out_e.reshape(_L, _B, _EPAD)[:, :, :_EDIM],
    ], axis=2)


# speedup vs baseline: 1.6054x; 1.2439x over previous
import functools

import jax
import jax.numpy as jnp
from jax import lax
from jax.experimental import pallas as pl
from jax.experimental.pallas import tpu as pltpu
from jax.experimental.pallas import tpu_sc as plsc

_VOCAB = 100000
_DIM = 64
_B = 4096
_L = 200
_TAGS = 59
_TDIM = 4
_EDIM = _TDIM + 2
_EPAD = 8
_ODIM = _DIM + _EDIM
_EXT = _TAGS * 4

_N = _B * _L
_LANES = 256
_ROWS = _N // _LANES
_NC = 2
_NS = 16
_NW = _NC * _NS
_ROWS_PER_W = _ROWS // _NW
_CHUNK_ROWS = 2
_CHUNK = _CHUNK_ROWS * _LANES
_STEPS = _ROWS_PER_W // _CHUNK_ROWS


def _sc_gather(word_table, ext_table, idx_all):
    mesh = plsc.VectorSubcoreMesh(core_axis_name="c", subcore_axis_name="s")

    @functools.partial(
        pl.kernel,
        mesh=mesh,
        compiler_params=pltpu.CompilerParams(
            use_tc_tiling_on_sc=False, needs_layout_passes=False),
        out_type=[jax.ShapeDtypeStruct((_N, _DIM), jnp.float32),
                  jax.ShapeDtypeStruct((_N * _EPAD,), jnp.float32)],
        scratch_types=[
            pltpu.VMEM((_CHUNK_ROWS, 2, _LANES), jnp.int32),
            pltpu.VMEM((_CHUNK_ROWS, 2, _LANES), jnp.int32),
            pltpu.VMEM((_CHUNK, _DIM), jnp.float32),
            pltpu.VMEM((_CHUNK, _DIM), jnp.float32),
            pltpu.VMEM((_CHUNK * _EPAD,), jnp.float32),
            pltpu.VMEM((_CHUNK * _EPAD,), jnp.float32),
            pltpu.VMEM((_EXT * _EPAD,), jnp.float32),
            pltpu.SemaphoreType.DMA,
            pltpu.SemaphoreType.DMA,
            pltpu.SemaphoreType.DMA,
            pltpu.SemaphoreType.DMA,
        ],
    )
    def k(word_hbm, ext_hbm, idx_hbm, outw_hbm, oute_hbm,
          ibuf0, ibuf1, wbuf0, wbuf1, ebuf0, ebuf1, extv,
          gsem0, gsem1, ssem0, ssem1):
        wid = lax.axis_index("s") * _NC + lax.axis_index("c")
        row0 = wid * _ROWS_PER_W
        ibuf = (ibuf0, ibuf1)
        wbuf = (wbuf0, wbuf1)
        ebuf = (ebuf0, ebuf1)
        gsem = (gsem0, gsem1)
        ssem = (ssem0, ssem1)

        def gather_copies(p):
            for b in range(_CHUNK_ROWS):
                yield pltpu.make_async_copy(
                    word_hbm.at[ibuf[p].at[b, 0]],
                    wbuf[p].at[pl.ds(b * _LANES, _LANES)],
                    gsem[p])

        def ext_compute(p):
            lane = lax.iota(jnp.int32, 16)
            for b in range(_CHUNK_ROWS):
                for g in range(_LANES // 16):
                    i0 = b * _LANES + g * 16
                    e8 = ibuf[p][b, 1, pl.ds(g * 16, 16)] * _EPAD
                    pos8 = (lane + i0) * _EPAD
                    for c in range(_EDIM):
                        vals = plsc.load_gather(extv, [e8 + c])
                        plsc.store_scatter(ebuf[p], [pos8 + c], vals)

        def store_copies(p, r):
            base = r * _LANES
            yield pltpu.make_async_copy(
                wbuf[p], outw_hbm.at[pl.ds(base, _CHUNK)], ssem[p])
            yield pltpu.make_async_copy(
                ebuf[p], oute_hbm.at[pl.ds(base * _EPAD, _CHUNK * _EPAD)],
                ssem[p])

        pltpu.sync_copy(ext_hbm, extv)
        pltpu.sync_copy(idx_hbm.at[pl.ds(row0, _CHUNK_ROWS)], ibuf[0])
        for c in gather_copies(0):
            c.start()
        pltpu.sync_copy(
            idx_hbm.at[pl.ds(row0 + _CHUNK_ROWS, _CHUNK_ROWS)], ibuf[1])

        def step(j, p):
            r = row0 + j * _CHUNK_ROWS

            @pl.when(j >= 1)
            def _():
                for c in store_copies(1 - p, r):
                    c.wait()

            ext_compute(p)

            for c in gather_copies(p):
                c.wait()

            @pl.when(j + 1 < _STEPS)
            def _():
                for c in gather_copies(1 - p):
                    c.start()

            for c in store_copies(p, r):
                c.start()

            @pl.when(j + 2 < _STEPS)
            def _():
                pltpu.sync_copy(
                    idx_hbm.at[pl.ds(r + 2 * _CHUNK_ROWS, _CHUNK_ROWS)],
                    ibuf[p])

        def body(i, carry):
            step(2 * i, 0)
            step(2 * i + 1, 1)
            return carry

        lax.fori_loop(0, _STEPS // 2, body, 0)

        for c in store_copies(1, row0 + (_STEPS - 1) * _CHUNK_ROWS):
            c.wait()

    return k(word_table, ext_table, idx_all)


def _tc_prep(data_0, data_1, data_2, data_3):
    bb = _LANES

    def body(d0, d1, d2, d3, o):
        e = d1[...] * 4 + d2[...] * 2 + d3[...]
        o[:, 0, 0, :] = jnp.transpose(d0[...], (1, 0))
        o[:, 0, 1, :] = jnp.transpose(e, (1, 0))

    out = pl.pallas_call(
        body,
        grid=(_B // bb,),
        in_specs=[pl.BlockSpec((bb, _L), lambda j: (j, 0))] * 4,
        out_specs=pl.BlockSpec((_L, 1, 2, _LANES), lambda j: (0, j, 0, 0)),
        out_shape=jax.ShapeDtypeStruct((_L, _B // bb, 2, _LANES), jnp.int32),
    )(data_0, data_1, data_2, data_3)
    return out.reshape(_ROWS, 2, _LANES)


def kernel(data_0, data_1, data_2, data_3, word_table, tag_table, is_content):
    s = jnp.asarray(is_content, jnp.float32)
    idx_all = _tc_prep(data_0, data_1, data_2, data_3)
    e = jnp.arange(_EXT, dtype=jnp.int32)
    ext = jnp.concatenate([
        jnp.repeat(tag_table, 4, axis=0),
        (((e >> 1) & 1).astype(jnp.float32) * s)[:, None],
        ((e & 1).astype(jnp.float32) * s)[:, None],
        jnp.zeros((_EXT, _EPAD - _EDIM), jnp.float32),
    ], axis=1)
    out_w, out_e = _sc_gather(word_table, ext.reshape(-1), idx_all)
    return jnp.concatenate([
        out_w.reshape(_L, _B, _DIM),
        out_e.reshape(_L, _B, _EPAD)[:, :, :_EDIM],
    ], axis=2)
